# baseline (speedup 1.0000x reference)
import functools

import jax
import jax.numpy as jnp
from jax.experimental import pallas as pl
from jax.experimental.pallas import tpu as pltpu
from jax.experimental.pallas import tpu_sc as plsc

NC = 2
NS = 16
NW = NC * NS
CHUNK = 128
NBUF = 4
D = 128
BM = 1000


def _sc_degree(dst2d, ones_hbm, zeros_hbm, acc_rows, rpt, cpw):
    mesh = plsc.VectorSubcoreMesh(core_axis_name="c", subcore_axis_name="s")

    @functools.partial(
        pl.kernel,
        out_type=jax.ShapeDtypeStruct((NC, acc_rows, D), jnp.float32),
        mesh=mesh,
        scratch_types=[
            pltpu.VMEM_SHARED((acc_rows, D), jnp.float32),
            pltpu.VMEM((CHUNK, D), jnp.float32),
            pltpu.VMEM((cpw, CHUNK), jnp.int32),
            pltpu.SemaphoreType.DMA((NBUF,)),
        ],
    )
    def k(dst_hbm, ones_h, zeros_h, out_hbm, acc, ones_v, didx_v, sems):
        cid = jax.lax.axis_index("c")
        sid = jax.lax.axis_index("s")
        wid = cid * NS + sid
        r0 = sid * rpt
        pltpu.sync_copy(zeros_h.at[pl.ds(r0, rpt)], acc.at[pl.ds(r0, rpt)])
        pltpu.sync_copy(ones_h, ones_v)
        pltpu.sync_copy(dst_hbm.at[pl.ds(wid * cpw, cpw)], didx_v)
        plsc.subcore_barrier()

        @pl.loop(0, cpw // NBUF)
        def _(it):
            c0 = it * NBUF
            for s in range(NBUF):
                pltpu.async_copy(ones_v, acc.at[didx_v.at[c0 + s]],
                                 sems.at[s], add=True)
            for s in range(NBUF):
                pltpu.make_async_copy(ones_v, acc.at[didx_v.at[c0 + s]],
                                      sems.at[s]).wait()

        plsc.subcore_barrier()
        pltpu.sync_copy(acc.at[pl.ds(r0, rpt)],
                        out_hbm.at[cid, pl.ds(r0, rpt)])

    return k(dst2d, ones_hbm, zeros_hbm)


def _sc_gather_scatter(table, src2d, dst2d, zeros_hbm, acc_rows, rpt, cpw):
    mesh = plsc.VectorSubcoreMesh(core_axis_name="c", subcore_axis_name="s")
    half = cpw // 2

    @functools.partial(
        pl.kernel,
        out_type=jax.ShapeDtypeStruct((NC, acc_rows, D), jnp.float32),
        mesh=mesh,
        scratch_types=[
            pltpu.VMEM_SHARED((acc_rows, D), jnp.float32),
            pltpu.VMEM((2, CHUNK, D), jnp.float32),
            pltpu.VMEM((half, CHUNK), jnp.int32),
            pltpu.VMEM((half, CHUNK), jnp.int32),
            pltpu.SemaphoreType.DMA((2,)),
            pltpu.SemaphoreType.DMA((2,)),
        ],
    )
    def k(tab_hbm, src_hbm, dst_hbm, zeros_h, out_hbm,
          acc, rows_v, sidx_v, didx_v, gsem, ssem):
        cid = jax.lax.axis_index("c")
        sid = jax.lax.axis_index("s")
        wid = cid * NS + sid
        r0 = sid * rpt
        pltpu.sync_copy(zeros_h.at[pl.ds(r0, rpt)], acc.at[pl.ds(r0, rpt)])
        plsc.subcore_barrier()

        for h in range(2):
            hb = wid * cpw + h * half
            pltpu.sync_copy(src_hbm.at[pl.ds(hb, half)], sidx_v)
            pltpu.sync_copy(dst_hbm.at[pl.ds(hb, half)], didx_v)
            for s in range(2):
                pltpu.async_copy(tab_hbm.at[sidx_v.at[s]], rows_v.at[s],
                                 gsem.at[s])

            @pl.loop(0, half // 2 - 1)
            def _(it):
                c0 = it * 2
                for s in range(2):
                    c = c0 + s
                    pltpu.make_async_copy(tab_hbm.at[sidx_v.at[c]],
                                          rows_v.at[s], gsem.at[s]).wait()
                    pltpu.async_copy(rows_v.at[s], acc.at[didx_v.at[c]],
                                     ssem.at[s], add=True)
                    pltpu.make_async_copy(rows_v.at[s],
                                          acc.at[didx_v.at[c]],
                                          ssem.at[s]).wait()
                    pltpu.async_copy(tab_hbm.at[sidx_v.at[c + 2]],
                                     rows_v.at[s], gsem.at[s])

            for s in range(2):
                c = half - 2 + s
                pltpu.make_async_copy(tab_hbm.at[sidx_v.at[c]],
                                      rows_v.at[s], gsem.at[s]).wait()
                pltpu.async_copy(rows_v.at[s], acc.at[didx_v.at[c]],
                                 ssem.at[s], add=True)
                pltpu.make_async_copy(rows_v.at[s], acc.at[didx_v.at[c]],
                                      ssem.at[s]).wait()

        plsc.subcore_barrier()
        pltpu.sync_copy(acc.at[pl.ds(r0, rpt)],
                        out_hbm.at[cid, pl.ds(r0, rpt)])

    return k(table, src2d, dst2d, zeros_hbm)


def _mm(x, w, dinv=None):
    m, kdim = x.shape
    n = w.shape[1]
    in_specs = [
        pl.BlockSpec((BM, kdim), lambda i: (i, 0)),
        pl.BlockSpec((kdim, n), lambda i: (0, 0)),
    ]
    args = [x, w]
    if dinv is not None:
        in_specs.append(pl.BlockSpec((BM, 1), lambda i: (i, 0)))
        args.append(dinv)

    def body(x_ref, w_ref, *rest):
        if dinv is not None:
            d_ref, o_ref = rest
        else:
            (o_ref,) = rest
        acc = jax.lax.dot_general(
            x_ref[...], w_ref[...], (((1,), (0,)), ((), ())),
            preferred_element_type=jnp.float32,
            precision=jax.lax.Precision.HIGHEST)
        if dinv is not None:
            acc = acc * d_ref[...]
        o_ref[...] = acc

    return pl.pallas_call(
        body, grid=(m // BM,), in_specs=in_specs,
        out_specs=pl.BlockSpec((BM, n), lambda i: (i, 0)),
        out_shape=jax.ShapeDtypeStruct((m, n), jnp.float32))(*args)


def _prep(degp, xw):
    m = xw.shape[0]

    def body(dp_ref, xw_ref, dinv_ref, y_ref):
        deg = dp_ref[0, :, 0:1] + dp_ref[1, :, 0:1] + 1.0
        dinv = jax.lax.rsqrt(deg)
        dinv_ref[...] = dinv
        y_ref[...] = xw_ref[...] * dinv

    return pl.pallas_call(
        body, grid=(m // BM,),
        in_specs=[
            pl.BlockSpec((NC, BM, D), lambda i: (0, i, 0)),
            pl.BlockSpec((BM, D), lambda i: (i, 0)),
        ],
        out_specs=[
            pl.BlockSpec((BM, 1), lambda i: (i, 0)),
            pl.BlockSpec((BM, D), lambda i: (i, 0)),
        ],
        out_shape=[
            jax.ShapeDtypeStruct((m, 1), jnp.float32),
            jax.ShapeDtypeStruct((m, D), jnp.float32),
        ])(degp, xw)


def _combine(sp, y, dinv, b, apply_tanh):
    m = y.shape[0]

    def body(sp_ref, y_ref, d_ref, b_ref, o_ref):
        z = (sp_ref[0] + sp_ref[1] + y_ref[...]) * d_ref[...] + b_ref[...]
        o_ref[...] = jnp.tanh(z) if apply_tanh else z

    return pl.pallas_call(
        body, grid=(m // BM,),
        in_specs=[
            pl.BlockSpec((NC, BM, D), lambda i: (0, i, 0)),
            pl.BlockSpec((BM, D), lambda i: (i, 0)),
            pl.BlockSpec((BM, 1), lambda i: (i, 0)),
            pl.BlockSpec((1, D), lambda i: (0, 0)),
        ],
        out_specs=pl.BlockSpec((BM, D), lambda i: (i, 0)),
        out_shape=jax.ShapeDtypeStruct((m, D), jnp.float32))(sp, y, dinv, b)


def kernel(x, edge_index, W1, b1, W2, b2):
    n = x.shape[0]
    e = edge_index.shape[1]
    src = edge_index[0].astype(jnp.int32)
    dst = edge_index[1].astype(jnp.int32)

    epg = NW * CHUNK * NBUF
    ep = ((e + epg - 1) // epg) * epg
    cpw = ep // (NW * CHUNK)
    if ep != e:
        src = jnp.concatenate([src, jnp.zeros((ep - e,), jnp.int32)])
        dst = jnp.concatenate([dst, jnp.full((ep - e,), n, jnp.int32)])
    src2d = src.reshape(ep // CHUNK, CHUNK)
    dst2d = dst.reshape(ep // CHUNK, CHUNK)

    acc_rows = ((n + 1 + NS * 8 - 1) // (NS * 8)) * (NS * 8)
    rpt = acc_rows // NS

    ones128 = jnp.ones((CHUNK, D), jnp.float32)
    zeros128 = jnp.zeros((acc_rows, D), jnp.float32)

    degp = _sc_degree(dst2d, ones128, zeros128, acc_rows, rpt, cpw)
    xw1 = _mm(x, W1)
    dinv, y1 = _prep(degp, xw1)
    s1 = _sc_gather_scatter(y1, src2d, dst2d, zeros128, acc_rows, rpt, cpw)
    h = _combine(s1, y1, dinv, b1.reshape(1, D), True)
    y2 = _mm(h, W2, dinv)
    s2 = _sc_gather_scatter(y2, src2d, dst2d, zeros128, acc_rows, rpt, cpw)
    out = _combine(s2, y2, dinv, b2.reshape(1, D), False)
    return out


# speedup vs baseline: 1.1635x; 1.1635x over previous
import functools

import jax
import jax.numpy as jnp
from jax.experimental import pallas as pl
from jax.experimental.pallas import tpu as pltpu
from jax.experimental.pallas import tpu_sc as plsc

NC = 2
NS = 16
NW = NC * NS
CHUNK = 128
NBUF = 4
D = 128
BM = 1000


def _sc_degree(dst2d, ones_hbm, zeros_hbm, acc_rows, rpt, cpw):
    mesh = plsc.VectorSubcoreMesh(core_axis_name="c", subcore_axis_name="s")

    @functools.partial(
        pl.kernel,
        out_type=jax.ShapeDtypeStruct((NC, acc_rows, D), jnp.float32),
        mesh=mesh,
        scratch_types=[
            pltpu.VMEM_SHARED((acc_rows, D), jnp.float32),
            pltpu.VMEM((CHUNK, D), jnp.float32),
            pltpu.VMEM((cpw, CHUNK), jnp.int32),
            pltpu.SemaphoreType.DMA((NBUF,)),
        ],
    )
    def k(dst_hbm, ones_h, zeros_h, out_hbm, acc, ones_v, didx_v, sems):
        cid = jax.lax.axis_index("c")
        sid = jax.lax.axis_index("s")
        wid = cid * NS + sid
        r0 = sid * rpt
        pltpu.sync_copy(zeros_h.at[pl.ds(r0, rpt)], acc.at[pl.ds(r0, rpt)])
        pltpu.sync_copy(ones_h, ones_v)
        pltpu.sync_copy(dst_hbm.at[pl.ds(wid * cpw, cpw)], didx_v)
        plsc.subcore_barrier()

        @pl.loop(0, cpw // NBUF)
        def _(it):
            c0 = it * NBUF
            for s in range(NBUF):
                pltpu.async_copy(ones_v, acc.at[didx_v.at[c0 + s]],
                                 sems.at[s], add=True)
            for s in range(NBUF):
                pltpu.make_async_copy(ones_v, acc.at[didx_v.at[c0 + s]],
                                      sems.at[s]).wait()

        plsc.subcore_barrier()
        pltpu.sync_copy(acc.at[pl.ds(r0, rpt)],
                        out_hbm.at[cid, pl.ds(r0, rpt)])

    return k(dst2d, ones_hbm, zeros_hbm)


def _sc_gather_scatter(table, src2d, dst2d, zeros_hbm, acc_rows, rpt,
                       cpw0, cpw1):
    mesh = plsc.VectorSubcoreMesh(core_axis_name="c", subcore_axis_name="s")
    hmax = max(cpw0, cpw1) // 2

    @functools.partial(
        pl.kernel,
        out_type=jax.ShapeDtypeStruct((NC, acc_rows, D), jnp.float32),
        mesh=mesh,
        scratch_types=[
            pltpu.VMEM_SHARED((acc_rows, D), jnp.float32),
            pltpu.VMEM((2, CHUNK, D), jnp.float32),
            pltpu.VMEM((hmax, CHUNK), jnp.int32),
            pltpu.VMEM((hmax, CHUNK), jnp.int32),
            pltpu.SemaphoreType.DMA((2,)),
            pltpu.SemaphoreType.DMA((2,)),
        ],
    )
    def k(tab_hbm, src_hbm, dst_hbm, zeros_h, out_hbm,
          acc, rows_v, sidx_v, didx_v, gsem, ssem):
        cid = jax.lax.axis_index("c")
        sid = jax.lax.axis_index("s")
        r0 = sid * rpt
        pltpu.sync_copy(zeros_h.at[pl.ds(r0, rpt)], acc.at[pl.ds(r0, rpt)])
        plsc.subcore_barrier()

        half_c = jnp.where(cid == 0, cpw0 // 2, cpw1 // 2)
        base_c = jnp.where(cid == 0, sid * cpw0, NS * cpw0 + sid * cpw1)

        for h in range(2):
            hb = base_c + h * half_c
            pltpu.sync_copy(src_hbm.at[pl.ds(hb, hmax)], sidx_v)
            pltpu.sync_copy(dst_hbm.at[pl.ds(hb, hmax)], didx_v)
            for s in range(2):
                pltpu.async_copy(tab_hbm.at[sidx_v.at[s]], rows_v.at[s],
                                 gsem.at[s])

            @pl.loop(0, hmax // 2)
            def _(it):
                c0 = it * 2
                for s in range(2):
                    c = c0 + s

                    @pl.when(c < half_c)
                    def _():
                        pltpu.make_async_copy(tab_hbm.at[sidx_v.at[c]],
                                              rows_v.at[s],
                                              gsem.at[s]).wait()
                        pltpu.async_copy(rows_v.at[s], acc.at[didx_v.at[c]],
                                         ssem.at[s], add=True)
                        pltpu.make_async_copy(rows_v.at[s],
                                              acc.at[didx_v.at[c]],
                                              ssem.at[s]).wait()

                        @pl.when(c + 2 < half_c)
                        def _():
                            pltpu.async_copy(tab_hbm.at[sidx_v.at[c + 2]],
                                             rows_v.at[s], gsem.at[s])

        plsc.subcore_barrier()
        pltpu.sync_copy(acc.at[pl.ds(r0, rpt)],
                        out_hbm.at[cid, pl.ds(r0, rpt)])

    return k(table, src2d, dst2d, zeros_hbm)


def _mm(x, w, dinv=None):
    m, kdim = x.shape
    n = w.shape[1]
    in_specs = [
        pl.BlockSpec((BM, kdim), lambda i: (i, 0)),
        pl.BlockSpec((kdim, n), lambda i: (0, 0)),
    ]
    args = [x, w]
    if dinv is not None:
        in_specs.append(pl.BlockSpec((BM, 1), lambda i: (i, 0)))
        args.append(dinv)

    def body(x_ref, w_ref, *rest):
        if dinv is not None:
            d_ref, o_ref = rest
        else:
            (o_ref,) = rest
        acc = jax.lax.dot_general(
            x_ref[...], w_ref[...], (((1,), (0,)), ((), ())),
            preferred_element_type=jnp.float32,
            precision=jax.lax.Precision.HIGHEST)
        if dinv is not None:
            acc = acc * d_ref[...]
        o_ref[...] = acc

    return pl.pallas_call(
        body, grid=(m // BM,), in_specs=in_specs,
        out_specs=pl.BlockSpec((BM, n), lambda i: (i, 0)),
        out_shape=jax.ShapeDtypeStruct((m, n), jnp.float32))(*args)


def _prep(degp, xw):
    m = xw.shape[0]

    def body(dp_ref, xw_ref, dinv_ref, y_ref):
        deg = dp_ref[0, :, 0:1] + dp_ref[1, :, 0:1] + 1.0
        dinv = jax.lax.rsqrt(deg)
        dinv_ref[...] = dinv
        y_ref[...] = xw_ref[...] * dinv

    return pl.pallas_call(
        body, grid=(m // BM,),
        in_specs=[
            pl.BlockSpec((NC, BM, D), lambda i: (0, i, 0)),
            pl.BlockSpec((BM, D), lambda i: (i, 0)),
        ],
        out_specs=[
            pl.BlockSpec((BM, 1), lambda i: (i, 0)),
            pl.BlockSpec((BM, D), lambda i: (i, 0)),
        ],
        out_shape=[
            jax.ShapeDtypeStruct((m, 1), jnp.float32),
            jax.ShapeDtypeStruct((m, D), jnp.float32),
        ])(degp, xw)


def _combine(sp, y, dinv, b, apply_tanh):
    m = y.shape[0]

    def body(sp_ref, y_ref, d_ref, b_ref, o_ref):
        z = (sp_ref[0] + sp_ref[1] + y_ref[...]) * d_ref[...] + b_ref[...]
        o_ref[...] = jnp.tanh(z) if apply_tanh else z

    return pl.pallas_call(
        body, grid=(m // BM,),
        in_specs=[
            pl.BlockSpec((NC, BM, D), lambda i: (0, i, 0)),
            pl.BlockSpec((BM, D), lambda i: (i, 0)),
            pl.BlockSpec((BM, 1), lambda i: (i, 0)),
            pl.BlockSpec((1, D), lambda i: (0, 0)),
        ],
        out_specs=pl.BlockSpec((BM, D), lambda i: (i, 0)),
        out_shape=jax.ShapeDtypeStruct((m, D), jnp.float32))(sp, y, dinv, b)


def kernel(x, edge_index, W1, b1, W2, b2):
    n = x.shape[0]
    e = edge_index.shape[1]
    src = edge_index[0].astype(jnp.int32)
    dst = edge_index[1].astype(jnp.int32)

    epg = NW * CHUNK * NBUF
    ep = ((e + epg - 1) // epg) * epg
    cpw = ep // (NW * CHUNK)
    cpw1 = max(4, (2 * cpw) // 5) & ~1
    cpw0 = 2 * cpw - cpw1
    hmax = cpw0 // 2
    ep_all = ep + hmax * CHUNK
    src = jnp.concatenate(
        [src, jnp.zeros((ep_all - e,), jnp.int32)])
    dst = jnp.concatenate(
        [dst, jnp.full((ep_all - e,), n, jnp.int32)])
    src2d = src.reshape(ep_all // CHUNK, CHUNK)
    dst2d = dst.reshape(ep_all // CHUNK, CHUNK)

    acc_rows = ((n + 1 + NS * 8 - 1) // (NS * 8)) * (NS * 8)
    rpt = acc_rows // NS

    ones128 = jnp.ones((CHUNK, D), jnp.float32)
    zeros128 = jnp.zeros((acc_rows, D), jnp.float32)

    degp = _sc_degree(dst2d, ones128, zeros128, acc_rows, rpt, cpw)
    xw1 = _mm(x, W1)
    dinv, y1 = _prep(degp, xw1)
    s1 = _sc_gather_scatter(y1, src2d, dst2d, zeros128, acc_rows, rpt,
                            cpw0, cpw1)
    h = _combine(s1, y1, dinv, b1.reshape(1, D), True)
    y2 = _mm(h, W2, dinv)
    s2 = _sc_gather_scatter(y2, src2d, dst2d, zeros128, acc_rows, rpt,
                            cpw0, cpw1)
    out = _combine(s2, y2, dinv, b2.reshape(1, D), False)
    return out
